# baseline (speedup 1.0000x reference)
import functools

import jax
import jax.numpy as jnp
from jax import lax
from jax.experimental import pallas as pl
from jax.experimental.pallas import tpu as pltpu
from jax.experimental.pallas import tpu_sc as plsc

N = 100000
D = 512
CH = 80
NCH = N // CH
NC = 2
NS = 16
NW = NC * NS

_mesh = plsc.VectorSubcoreMesh(core_axis_name="c", subcore_axis_name="s")


@functools.partial(
    pl.kernel,
    out_type=jax.ShapeDtypeStruct((N, D), jnp.float32),
    mesh=_mesh,
    scratch_types=[
        pltpu.VMEM((2, CH), jnp.int32),
        pltpu.VMEM((2, CH, D), jnp.float32),
        pltpu.SemaphoreType.DMA((2,)),
        pltpu.SemaphoreType.DMA((2,)),
    ],
)
def _emb_lookup(idx_hbm, table_hbm, out_hbm, idx_v, rows_v, gsem, ssem):
    wid = lax.axis_index("s") * NC + lax.axis_index("c")
    nchunks = (NCH - wid + NW - 1) // NW

    def base_of(i):
        return (wid + i * NW) * CH

    def start_gather(i, b):
        pltpu.sync_copy(idx_hbm.at[pl.ds(base_of(i), CH)], idx_v.at[b])
        pltpu.make_async_copy(
            table_hbm.at[idx_v.at[b]], rows_v.at[b], gsem.at[b]
        ).start()

    def wait_gather(b):
        pltpu.make_async_copy(
            out_hbm.at[pl.ds(0, CH)], rows_v.at[b], gsem.at[b]
        ).wait()

    def start_store(i, b):
        pltpu.make_async_copy(
            rows_v.at[b], out_hbm.at[pl.ds(base_of(i), CH)], ssem.at[b]
        ).start()

    def wait_store(b):
        pltpu.make_async_copy(
            rows_v.at[b], out_hbm.at[pl.ds(0, CH)], ssem.at[b]
        ).wait()

    start_gather(0, 0)
    start_gather(1, 1)

    npairs = nchunks // 2

    def body(g, carry):
        for b in (0, 1):
            i = 2 * g + b
            wait_gather(b)
            start_store(i, b)

            @pl.when(2 * (g + 1) + b < nchunks)
            def _():
                wait_store(b)
                start_gather(2 * (g + 1) + b, b)

        return carry

    lax.fori_loop(0, npairs, body, 0)

    @pl.when(nchunks % 2 == 1)
    def _():
        wait_gather(0)
        start_store(nchunks - 1, 0)

    wait_store(0)
    wait_store(1)


def kernel(x, W0):
    idx = x.reshape(N).astype(jnp.int32)
    return _emb_lookup(idx, W0)
